# baseline (speedup 1.0000x reference)
import functools

import jax
import jax.numpy as jnp
from jax import lax
from jax.experimental import pallas as pl
from jax.experimental.pallas import tpu as pltpu
from jax.experimental.pallas import tpu_sc as plsc

VOCAB = 100000
D = 1024
B = 4
S = 4096
T = B * S

NC = 2
NS = 16
NW = NC * NS
TPW = T // NW
C = 32
LANES = 16


def _body(ids_hbm, pos_hbm, wte_hbm, wpe_hbm, out_hbm,
          idx_tok, idx_pos, rows_a, rows_b, sem_a, sem_b):
    wid = lax.axis_index("s") * NC + lax.axis_index("c")
    base = wid * TPW
    pltpu.sync_copy(ids_hbm.at[pl.ds(base, TPW)], idx_tok)
    pltpu.sync_copy(pos_hbm.at[pl.ds(base, TPW)], idx_pos)

    def chunk_body(ci, carry):
        c0 = ci * C
        ca = pltpu.async_copy(wte_hbm.at[idx_tok.at[pl.ds(c0, C)]], rows_a, sem_a)
        cb = pltpu.async_copy(wpe_hbm.at[idx_pos.at[pl.ds(c0, C)]], rows_b, sem_b)
        ca.wait()
        cb.wait()

        def row_body(r, carry2):
            def col_body(j, carry3):
                off = pl.multiple_of(j * LANES, LANES)
                rows_a[r, pl.ds(off, LANES)] = (
                    rows_a[r, pl.ds(off, LANES)] + rows_b[r, pl.ds(off, LANES)]
                )
                return carry3
            return lax.fori_loop(0, D // LANES, col_body, carry2)

        lax.fori_loop(0, C, row_body, carry)
        pltpu.sync_copy(rows_a, out_hbm.at[pl.ds(base + c0, C)])
        return carry

    lax.fori_loop(0, TPW // C, chunk_body, 0)


_embed_call = functools.partial(
    pl.kernel,
    out_type=jax.ShapeDtypeStruct((T, D), jnp.float32),
    mesh=plsc.VectorSubcoreMesh(core_axis_name="c", subcore_axis_name="s"),
    scratch_types=[
        pltpu.VMEM((TPW,), jnp.int32),
        pltpu.VMEM((TPW,), jnp.int32),
        pltpu.VMEM((C, D), jnp.float32),
        pltpu.VMEM((C, D), jnp.float32),
        pltpu.SemaphoreType.DMA,
        pltpu.SemaphoreType.DMA,
    ],
)(_body)


def kernel(input_ids, position_ids, wte, wpe):
    ids = input_ids.reshape(T).astype(jnp.int32)
    pos = position_ids.reshape(T).astype(jnp.int32)
    out = _embed_call(ids, pos, wte, wpe)
    return out.reshape(B, S, D)


# speedup vs baseline: 2.6044x; 2.6044x over previous
import functools

import jax
import jax.numpy as jnp
from jax import lax
from jax.experimental import pallas as pl
from jax.experimental.pallas import tpu as pltpu
from jax.experimental.pallas import tpu_sc as plsc

VOCAB = 100000
D = 1024
B = 4
S = 4096
T = B * S

NC = 2
NS = 16
NW = NC * NS
TPW = T // NW
C = 16
NCH = TPW // C
LANES = 16


def _body(ids_hbm, pos_hbm, wte_hbm, wpe_hbm, out_hbm,
          idx_tok, idx_pos, ra0, rb0, ro0, ra1, rb1, ro1,
          sg0, sg1, so0, so1):
    wid = lax.axis_index("s") * NC + lax.axis_index("c")
    base = wid * TPW
    pltpu.sync_copy(ids_hbm.at[pl.ds(base, TPW)], idx_tok)
    pltpu.sync_copy(pos_hbm.at[pl.ds(base, TPW)], idx_pos)

    RA = (ra0, ra1)
    RB = (rb0, rb1)
    RO = (ro0, ro1)
    SG = (sg0, sg1)
    SO = (so0, so1)

    def issue_gathers(ch, b):
        c0 = ch * C
        pltpu.async_copy(wte_hbm.at[idx_tok.at[pl.ds(c0, C)]], RA[b], SG[b])
        pltpu.async_copy(wpe_hbm.at[idx_pos.at[pl.ds(c0, C)]], RB[b], SG[b])

    issue_gathers(0, 0)
    issue_gathers(1, 1)

    NI = NCH // 2

    def it(i, carry):
        for b in (0, 1):
            ch = i * 2 + b
            pltpu.make_async_copy(wte_hbm.at[pl.ds(0, C)], RA[b], SG[b]).wait()
            pltpu.make_async_copy(wte_hbm.at[pl.ds(0, C)], RB[b], SG[b]).wait()

            @pl.when(i > 0)
            def _wait_out():
                pltpu.make_async_copy(
                    RO[b], out_hbm.at[pl.ds(0, C)], SO[b]).wait()

            def row_body(r, c2, _b=b):
                for j in range(D // LANES):
                    sl = pl.ds(j * LANES, LANES)
                    RO[_b][r, sl] = RA[_b][r, sl] + RB[_b][r, sl]
                return c2
            lax.fori_loop(0, C, row_body, 0)

            @pl.when(i < NI - 1)
            def _prefetch():
                issue_gathers(ch + 2, b)

            pltpu.async_copy(RO[b], out_hbm.at[pl.ds(base + ch * C, C)], SO[b])
        return carry

    lax.fori_loop(0, NI, it, 0)
    for b in (0, 1):
        pltpu.make_async_copy(RO[b], out_hbm.at[pl.ds(0, C)], SO[b]).wait()


_embed_call = functools.partial(
    pl.kernel,
    out_type=jax.ShapeDtypeStruct((T, D), jnp.float32),
    mesh=plsc.VectorSubcoreMesh(core_axis_name="c", subcore_axis_name="s"),
    scratch_types=[
        pltpu.VMEM((TPW,), jnp.int32),
        pltpu.VMEM((TPW,), jnp.int32),
        pltpu.VMEM((C, D), jnp.float32),
        pltpu.VMEM((C, D), jnp.float32),
        pltpu.VMEM((C, D), jnp.float32),
        pltpu.VMEM((C, D), jnp.float32),
        pltpu.VMEM((C, D), jnp.float32),
        pltpu.VMEM((C, D), jnp.float32),
        pltpu.SemaphoreType.DMA,
        pltpu.SemaphoreType.DMA,
        pltpu.SemaphoreType.DMA,
        pltpu.SemaphoreType.DMA,
    ],
)(_body)


def kernel(input_ids, position_ids, wte, wpe):
    ids = input_ids.reshape(T).astype(jnp.int32)
    pos = position_ids.reshape(T).astype(jnp.int32)
    out = _embed_call(ids, pos, wte, wpe)
    return out.reshape(B, S, D)
